# baseline (speedup 1.0000x reference)
import functools

import jax
import jax.numpy as jnp
from jax import lax
from jax.experimental import pallas as pl
from jax.experimental.pallas import tpu as pltpu
from jax.experimental.pallas import tpu_sc as plsc

N_NODES = 10000
N_PAD = 10240
D_FEAT = 128
D_EDGE = 16
N_HIDDEN = 32
D_TAB = 576

NC, NS = 2, 16
NW = NC * NS
CHUNK = 80
RPW = N_PAD // NS


def _table_body(x_ref, w_ref, y_ref):
    y_ref[...] = jnp.dot(x_ref[...], w_ref[...],
                         preferred_element_type=jnp.float32
                         ).astype(jnp.bfloat16)


def _make_table(x, w_all):
    blk = 2000
    return pl.pallas_call(
        _table_body,
        grid=(N_NODES // blk,),
        in_specs=[pl.BlockSpec((blk, D_FEAT), lambda i: (i, 0)),
                  pl.BlockSpec((D_FEAT, D_TAB), lambda i: (0, 0))],
        out_specs=pl.BlockSpec((blk, D_TAB), lambda i: (i, 0)),
        out_shape=jax.ShapeDtypeStruct((N_NODES, D_TAB), jnp.bfloat16),
    )(x, w_all)


def _edge_body(nchunk, table_hbm, src_hbm, dst_hbm, ea_hbm, zeros_hbm,
               agg_hbm, src_v, dst_v, ea_v, rows_v, msg_v, agg_sh,
               isem0, isem1, gsem0, gsem1):
    c = lax.axis_index("c")
    s = lax.axis_index("s")
    wid = c * NS + s
    row0 = wid * nchunk
    isem = (isem0, isem1)
    gsem = (gsem0, gsem1)
    npair = CHUNK // 2

    pltpu.sync_copy(zeros_hbm, agg_sh.at[pl.ds(s * RPW, RPW)])
    plsc.subcore_barrier()

    def idx_copies(g, b):
        r = row0 + g
        return (
            pltpu.make_async_copy(src_hbm.at[pl.ds(r * CHUNK, CHUNK)],
                                  src_v.at[b], isem[b]),
            pltpu.make_async_copy(dst_hbm.at[pl.ds(r * CHUNK, CHUNK)],
                                  dst_v.at[b], isem[b]),
            pltpu.make_async_copy(ea_hbm.at[pl.ds(r * CHUNK, CHUNK)],
                                  ea_v.at[b], isem[b]),
        )

    def start_idx(g, b):
        for cp in idx_copies(g, b):
            cp.start()

    def wait_idx(g, b):
        for cp in idx_copies(g, b):
            cp.wait()

    def gather_copy(b):
        return pltpu.make_async_copy(table_hbm.at[src_v.at[b]],
                                     rows_v.at[b], gsem[b])

    lanes = lax.iota(jnp.int32, 16)
    idx_even = jnp.minimum(lanes * 2, 15)
    idx_odd = jnp.minimum(lanes * 2 + 1, 15)
    _gdn = lax.GatherDimensionNumbers(
        offset_dims=(), collapsed_slice_dims=(0,), start_index_map=(0,))

    def lane_gather(v, idx):
        return lax.gather(v, idx[:, None], dimension_numbers=_gdn,
                          slice_sizes=(1,),
                          mode=lax.GatherScatterMode.PROMISE_IN_BOUNDS)

    def compute_scatter(b):
        @plsc.parallel_loop(0, CHUNK, step=1, unroll=8)
        def edge_body(e):
            ear = ea_v[b, e, :]
            wa = lane_gather(ear, idx_even)
            wb = lane_gather(ear, idx_odd)
            wall = plsc.pack(wa, wb, format=plsc.PackFormat.INTERLEAVED)
            wi = plsc.bitcast(wall, jnp.int32)
            acc0 = rows_v[b, e, pl.ds(512, 32)]
            acc1 = rows_v[b, e, pl.ds(544, 32)]
            for t in range(8):
                wv = plsc.bitcast(
                    jnp.broadcast_to(wi[t], (16,)), jnp.bfloat16)
                acc0 = acc0 + wv * rows_v[b, e, pl.ds(t * 64, 32)]
                acc1 = acc1 + wv * rows_v[b, e, pl.ds(t * 64 + 32, 32)]
            lo0, hi0 = plsc.unpack(acc0, format=plsc.PackFormat.INTERLEAVED)
            lo1, hi1 = plsc.unpack(acc1, format=plsc.PackFormat.INTERLEAVED)
            msg_v[e, pl.ds(0, 16)] = lo0 + hi0
            msg_v[e, pl.ds(16, 16)] = lo1 + hi1

        pltpu.sync_copy(msg_v, agg_sh.at[dst_v.at[b]], add=True)

    start_idx(0, 0)
    wait_idx(0, 0)
    gather_copy(0).start()
    start_idx(1, 1)

    def pair_body(p, carry):
        for b in (0, 1):
            g = 2 * p + b
            nb = 1 - b

            @pl.when(g + 1 < nchunk)
            def _():
                wait_idx(g + 1, nb)
                gather_copy(nb).start()

            gather_copy(b).wait()
            compute_scatter(b)

            @pl.when(g + 2 < nchunk)
            def _():
                start_idx(g + 2, b)
        return carry

    lax.fori_loop(0, nchunk // 2, pair_body, 0)
    if nchunk % 2:
        gather_copy(0).wait()
        compute_scatter(0)
    plsc.subcore_barrier()
    pltpu.sync_copy(agg_sh.at[pl.ds(s * RPW, RPW)],
                    agg_hbm.at[c, pl.ds(s * RPW, RPW)])


def _edge_call(table, src, dst, edge_attr):
    n_edges = src.shape[0]
    nchunk = n_edges // (NW * CHUNK)
    zeros = jnp.zeros((RPW, N_HIDDEN), jnp.float32)
    mesh = plsc.VectorSubcoreMesh(core_axis_name="c", subcore_axis_name="s")
    return pl.kernel(
        functools.partial(_edge_body, nchunk),
        out_type=jax.ShapeDtypeStruct((NC, N_PAD, N_HIDDEN), jnp.float32),
        mesh=mesh,
        compiler_params=pltpu.CompilerParams(use_tc_tiling_on_sc=False,
                                             needs_layout_passes=False),
        scratch_types=[
            pltpu.VMEM((2, CHUNK), jnp.int32),
            pltpu.VMEM((2, CHUNK), jnp.int32),
            pltpu.VMEM((2, CHUNK, D_EDGE), jnp.float32),
            pltpu.VMEM((2, CHUNK, D_TAB), jnp.bfloat16),
            pltpu.VMEM((CHUNK, N_HIDDEN), jnp.float32),
            pltpu.VMEM_SHARED((N_PAD, N_HIDDEN), jnp.float32),
            pltpu.SemaphoreType.DMA,
            pltpu.SemaphoreType.DMA,
            pltpu.SemaphoreType.DMA,
            pltpu.SemaphoreType.DMA,
        ],
    )(table, src, dst, edge_attr, zeros)


def _tail_body(agg_ref, x_ref, root_ref, bias_ref, gamma_ref, beta_ref,
               mean_ref, var_ref, dw_ref, db_ref, out_ref, acc_ref):
    i = pl.program_id(0)

    @pl.when(i == 0)
    def _():
        acc_ref[...] = jnp.zeros_like(acc_ref)

    h = (agg_ref[0] + agg_ref[1]
         + jnp.dot(x_ref[...], root_ref[...],
                   preferred_element_type=jnp.float32)
         + bias_ref[...])
    h = jnp.maximum(h, 0.0)
    acc_ref[...] += jnp.sum(h, axis=0, keepdims=True)

    @pl.when(i == pl.num_programs(0) - 1)
    def _():
        a = gamma_ref[...] * lax.rsqrt(var_ref[...] + 1e-3)
        pooled = (a * acc_ref[...]
                  + N_NODES * (beta_ref[...] - a * mean_ref[...]))
        row = jnp.dot(pooled, dw_ref[...],
                      preferred_element_type=jnp.float32) + db_ref[...]
        out_ref[...] = jnp.broadcast_to(row, out_ref.shape)


def _tail_call(agg2, x, root_kernel, bias, gamma, beta, mean, var, dw, db):
    blk = 2000
    grid = (N_NODES // blk,)
    vec = lambda: pl.BlockSpec((1, N_HIDDEN), lambda i: (0, 0))
    return pl.pallas_call(
        _tail_body,
        grid=grid,
        in_specs=[
            pl.BlockSpec((NC, blk, N_HIDDEN), lambda i: (0, i, 0)),
            pl.BlockSpec((blk, D_FEAT), lambda i: (i, 0)),
            pl.BlockSpec((D_FEAT, N_HIDDEN), lambda i: (0, 0)),
            vec(), vec(), vec(), vec(), vec(),
            pl.BlockSpec((N_HIDDEN, 128), lambda i: (0, 0)),
            pl.BlockSpec((1, 128), lambda i: (0, 0)),
        ],
        out_specs=pl.BlockSpec((8, 128), lambda i: (0, 0)),
        out_shape=jax.ShapeDtypeStruct((8, 128), jnp.float32),
        scratch_shapes=[pltpu.VMEM((1, N_HIDDEN), jnp.float32)],
    )(agg2, x, root_kernel, bias, gamma, beta, mean, var, dw, db)


def kernel(x, edge_index, edge_attr, Wk, bk, root_kernel, bias, gamma, beta,
           moving_mean, moving_var, dense_W, dense_b):
    wk4 = Wk.reshape(8, 2, D_FEAT, 2, 16)
    main = wk4.transpose(2, 0, 3, 4, 1).reshape(D_FEAT, 512)
    bk3 = bk.reshape(D_FEAT, 2, 16)
    biasblk = jnp.stack([bk3, jnp.zeros_like(bk3)], axis=-1
                        ).reshape(D_FEAT, 64)
    w_all = jnp.concatenate([main, biasblk], axis=1)
    table = _make_table(x, w_all)

    agg2 = _edge_call(table, edge_index[0], edge_index[1], edge_attr)
    dw_pad = jnp.zeros((N_HIDDEN, 128), jnp.float32).at[:, :3].set(dense_W)
    db_pad = jnp.zeros((1, 128), jnp.float32).at[0, :3].set(dense_b)
    r = lambda v: v.reshape(1, N_HIDDEN)
    outp = _tail_call(agg2, x, root_kernel, r(bias), r(gamma), r(beta),
                      r(moving_mean), r(moving_var), dw_pad, db_pad)
    return outp[0, :3]


# speedup vs baseline: 1.1945x; 1.1945x over previous
import functools

import jax
import jax.numpy as jnp
from jax import lax
from jax.experimental import pallas as pl
from jax.experimental.pallas import tpu as pltpu
from jax.experimental.pallas import tpu_sc as plsc

N_NODES = 10000
N_PAD = 10240
D_FEAT = 128
D_EDGE = 16
N_HIDDEN = 32
D_TAB = 576

NC, NS = 2, 16
NW = NC * NS
CHUNK = 80
RPW = N_PAD // NS


def _table_body(x_ref, w_ref, y_ref):
    y_ref[...] = jnp.dot(x_ref[...], w_ref[...],
                         preferred_element_type=jnp.float32
                         ).astype(jnp.bfloat16)


def _make_table(x, w_all):
    blk = 2000
    return pl.pallas_call(
        _table_body,
        grid=(N_NODES // blk,),
        in_specs=[pl.BlockSpec((blk, D_FEAT), lambda i: (i, 0)),
                  pl.BlockSpec((D_FEAT, D_TAB), lambda i: (0, 0))],
        out_specs=pl.BlockSpec((blk, D_TAB), lambda i: (i, 0)),
        out_shape=jax.ShapeDtypeStruct((N_NODES, D_TAB), jnp.bfloat16),
    )(x, w_all)


def _edge_body(nchunk, table_hbm, src_hbm, dst_hbm, ea_hbm, zeros_hbm,
               agg_hbm, src_v, dst_v, ea_v, rows_v, msg_v, agg_sh,
               isem0, isem1, gsem0, gsem1):
    c = lax.axis_index("c")
    s = lax.axis_index("s")
    wid = c * NS + s
    row0 = wid * nchunk
    isem = (isem0, isem1)
    gsem = (gsem0, gsem1)
    npair = CHUNK // 2

    pltpu.sync_copy(zeros_hbm, agg_sh.at[pl.ds(s * RPW, RPW)])
    plsc.subcore_barrier()

    def idx_copies(g, b):
        r = row0 + g
        return (
            pltpu.make_async_copy(src_hbm.at[pl.ds(r * CHUNK, CHUNK)],
                                  src_v.at[b], isem[b]),
            pltpu.make_async_copy(dst_hbm.at[pl.ds(r * CHUNK, CHUNK)],
                                  dst_v.at[b], isem[b]),
            pltpu.make_async_copy(ea_hbm.at[pl.ds(r * CHUNK, CHUNK)],
                                  ea_v.at[b], isem[b]),
        )

    def start_idx(g, b):
        for cp in idx_copies(g, b):
            cp.start()

    def wait_idx(g, b):
        for cp in idx_copies(g, b):
            cp.wait()

    def gather_copy(b):
        return pltpu.make_async_copy(table_hbm.at[src_v.at[b]],
                                     rows_v.at[b], gsem[b])

    lanes = lax.iota(jnp.int32, 16)
    idx_even = jnp.minimum(lanes * 2, 15)
    idx_odd = jnp.minimum(lanes * 2 + 1, 15)
    _gdn = lax.GatherDimensionNumbers(
        offset_dims=(), collapsed_slice_dims=(0,), start_index_map=(0,))

    def lane_gather(v, idx):
        return lax.gather(v, idx[:, None], dimension_numbers=_gdn,
                          slice_sizes=(1,),
                          mode=lax.GatherScatterMode.PROMISE_IN_BOUNDS)

    def compute_scatter(b):
        @plsc.parallel_loop(0, CHUNK, step=1, unroll=4)
        def edge_body(e):
            ear = ea_v[b, e, :]
            wa = lane_gather(ear, idx_even)
            wb = lane_gather(ear, idx_odd)
            wall = plsc.pack(wa, wb, format=plsc.PackFormat.INTERLEAVED)
            wi = plsc.bitcast(wall, jnp.int32)
            acc0 = rows_v[b, e, pl.ds(512, 32)]
            acc1 = rows_v[b, e, pl.ds(544, 32)]
            for t in range(8):
                wv = plsc.bitcast(
                    jnp.broadcast_to(wi[t], (16,)), jnp.bfloat16)
                acc0 = acc0 + wv * rows_v[b, e, pl.ds(t * 64, 32)]
                acc1 = acc1 + wv * rows_v[b, e, pl.ds(t * 64 + 32, 32)]
            lo0, hi0 = plsc.unpack(acc0, format=plsc.PackFormat.INTERLEAVED)
            lo1, hi1 = plsc.unpack(acc1, format=plsc.PackFormat.INTERLEAVED)
            msg_v[e, pl.ds(0, 16)] = lo0 + hi0
            msg_v[e, pl.ds(16, 16)] = lo1 + hi1

        pltpu.sync_copy(msg_v, agg_sh.at[dst_v.at[b]], add=True)

    start_idx(0, 0)
    wait_idx(0, 0)
    gather_copy(0).start()
    start_idx(1, 1)

    def pair_body(p, carry):
        for b in (0, 1):
            g = 2 * p + b
            nb = 1 - b

            @pl.when(g + 1 < nchunk)
            def _():
                wait_idx(g + 1, nb)
                gather_copy(nb).start()

            gather_copy(b).wait()
            compute_scatter(b)

            @pl.when(g + 2 < nchunk)
            def _():
                start_idx(g + 2, b)
        return carry

    lax.fori_loop(0, nchunk // 2, pair_body, 0)
    if nchunk % 2:
        gather_copy(0).wait()
        compute_scatter(0)
    plsc.subcore_barrier()
    pltpu.sync_copy(agg_sh.at[pl.ds(s * RPW, RPW)],
                    agg_hbm.at[c, pl.ds(s * RPW, RPW)])


def _edge_call(table, src, dst, edge_attr):
    n_edges = src.shape[0]
    nchunk = n_edges // (NW * CHUNK)
    zeros = jnp.zeros((RPW, N_HIDDEN), jnp.float32)
    mesh = plsc.VectorSubcoreMesh(core_axis_name="c", subcore_axis_name="s")
    return pl.kernel(
        functools.partial(_edge_body, nchunk),
        out_type=jax.ShapeDtypeStruct((NC, N_PAD, N_HIDDEN), jnp.float32),
        mesh=mesh,
        compiler_params=pltpu.CompilerParams(use_tc_tiling_on_sc=False,
                                             needs_layout_passes=False),
        scratch_types=[
            pltpu.VMEM((2, CHUNK), jnp.int32),
            pltpu.VMEM((2, CHUNK), jnp.int32),
            pltpu.VMEM((2, CHUNK, D_EDGE), jnp.float32),
            pltpu.VMEM((2, CHUNK, D_TAB), jnp.bfloat16),
            pltpu.VMEM((CHUNK, N_HIDDEN), jnp.float32),
            pltpu.VMEM_SHARED((N_PAD, N_HIDDEN), jnp.float32),
            pltpu.SemaphoreType.DMA,
            pltpu.SemaphoreType.DMA,
            pltpu.SemaphoreType.DMA,
            pltpu.SemaphoreType.DMA,
        ],
    )(table, src, dst, edge_attr, zeros)


def _tail_body(agg_ref, x_ref, root_ref, bias_ref, gamma_ref, beta_ref,
               mean_ref, var_ref, dw_ref, db_ref, out_ref, acc_ref):
    i = pl.program_id(0)

    @pl.when(i == 0)
    def _():
        acc_ref[...] = jnp.zeros_like(acc_ref)

    h = (agg_ref[0] + agg_ref[1]
         + jnp.dot(x_ref[...], root_ref[...],
                   preferred_element_type=jnp.float32)
         + bias_ref[...])
    h = jnp.maximum(h, 0.0)
    acc_ref[...] += jnp.sum(h, axis=0, keepdims=True)

    @pl.when(i == pl.num_programs(0) - 1)
    def _():
        a = gamma_ref[...] * lax.rsqrt(var_ref[...] + 1e-3)
        pooled = (a * acc_ref[...]
                  + N_NODES * (beta_ref[...] - a * mean_ref[...]))
        row = jnp.dot(pooled, dw_ref[...],
                      preferred_element_type=jnp.float32) + db_ref[...]
        out_ref[...] = jnp.broadcast_to(row, out_ref.shape)


def _tail_call(agg2, x, root_kernel, bias, gamma, beta, mean, var, dw, db):
    blk = 2000
    grid = (N_NODES // blk,)
    vec = lambda: pl.BlockSpec((1, N_HIDDEN), lambda i: (0, 0))
    return pl.pallas_call(
        _tail_body,
        grid=grid,
        in_specs=[
            pl.BlockSpec((NC, blk, N_HIDDEN), lambda i: (0, i, 0)),
            pl.BlockSpec((blk, D_FEAT), lambda i: (i, 0)),
            pl.BlockSpec((D_FEAT, N_HIDDEN), lambda i: (0, 0)),
            vec(), vec(), vec(), vec(), vec(),
            pl.BlockSpec((N_HIDDEN, 128), lambda i: (0, 0)),
            pl.BlockSpec((1, 128), lambda i: (0, 0)),
        ],
        out_specs=pl.BlockSpec((8, 128), lambda i: (0, 0)),
        out_shape=jax.ShapeDtypeStruct((8, 128), jnp.float32),
        scratch_shapes=[pltpu.VMEM((1, N_HIDDEN), jnp.float32)],
    )(agg2, x, root_kernel, bias, gamma, beta, mean, var, dw, db)


def kernel(x, edge_index, edge_attr, Wk, bk, root_kernel, bias, gamma, beta,
           moving_mean, moving_var, dense_W, dense_b):
    wk4 = Wk.reshape(8, 2, D_FEAT, 2, 16)
    main = wk4.transpose(2, 0, 3, 4, 1).reshape(D_FEAT, 512)
    bk3 = bk.reshape(D_FEAT, 2, 16)
    biasblk = jnp.stack([bk3, jnp.zeros_like(bk3)], axis=-1
                        ).reshape(D_FEAT, 64)
    w_all = jnp.concatenate([main, biasblk], axis=1)
    table = _make_table(x, w_all)

    agg2 = _edge_call(table, edge_index[0], edge_index[1], edge_attr)
    dw_pad = jnp.zeros((N_HIDDEN, 128), jnp.float32).at[:, :3].set(dense_W)
    db_pad = jnp.zeros((1, 128), jnp.float32).at[0, :3].set(dense_b)
    r = lambda v: v.reshape(1, N_HIDDEN)
    outp = _tail_call(agg2, x, root_kernel, r(bias), r(gamma), r(beta),
                      r(moving_mean), r(moving_var), dw_pad, db_pad)
    return outp[0, :3]
